# Initial kernel scaffold; baseline (speedup 1.0000x reference)
import jax
import jax.numpy as jnp
from jax.experimental import pallas as pl


def kernel(x, table):
    raise NotImplementedError("write your pallas kernel here")


# speedup vs baseline: 1.0232x; 1.0232x over previous
import functools

import jax
import jax.numpy as jnp
from jax import lax
from jax.experimental import pallas as pl
from jax.experimental.pallas import tpu as pltpu
from jax.experimental.pallas import tpu_sc as plsc

D = 32
G = 128
NC = 2
NS = 16


@functools.lru_cache(maxsize=None)
def _make_kernel(B, V):
    NW = NC * NS
    b_per_w = B // NW
    n_g = b_per_w // G
    mesh = plsc.VectorSubcoreMesh(core_axis_name="c", subcore_axis_name="s")

    @functools.partial(
        pl.kernel,
        out_type=jax.ShapeDtypeStruct((B, D), jnp.float32),
        mesh=mesh,
        scratch_types=[
            pltpu.VMEM((b_per_w,), jnp.int32),
            pltpu.VMEM((G, D), jnp.float32),
            pltpu.SemaphoreType.DMA,
        ],
        compiler_params=pltpu.CompilerParams(use_tc_tiling_on_sc=False),
    )
    def k(x_hbm, table_hbm, out_hbm, idx_v, rows_v, sem):
        wid = lax.axis_index("s") * NC + lax.axis_index("c")
        base = wid * b_per_w
        pltpu.sync_copy(x_hbm.at[pl.ds(base, b_per_w)], idx_v)

        def step(g, carry):
            off = pl.multiple_of(g * G, G)
            pltpu.async_copy(
                table_hbm.at[idx_v.at[pl.ds(off, G)]], rows_v, sem
            ).wait()
            pltpu.sync_copy(rows_v, out_hbm.at[pl.ds(base + off, G)])
            return carry

        lax.fori_loop(0, n_g, step, 0)

    return k


def kernel(x, table):
    lead = x.shape
    xf = x.reshape(-1).astype(jnp.int32)
    out = _make_kernel(xf.shape[0], table.shape[0])(xf, table)
    return out.reshape(*lead, D)


# speedup vs baseline: 1.1129x; 1.0876x over previous
import functools

import jax
import jax.numpy as jnp
from jax import lax
from jax.experimental import pallas as pl
from jax.experimental.pallas import tpu as pltpu
from jax.experimental.pallas import tpu_sc as plsc

D = 32
G = 128
NC = 2
NS = 16
R = 8


@functools.lru_cache(maxsize=None)
def _make_kernel(B, V):
    NW = NC * NS
    b_per_w = B // NW
    n_g = b_per_w // G
    n_blocks = n_g // R
    mesh = plsc.VectorSubcoreMesh(core_axis_name="c", subcore_axis_name="s")

    @functools.partial(
        pl.kernel,
        out_type=jax.ShapeDtypeStruct((B, D), jnp.float32),
        mesh=mesh,
        scratch_types=[
            pltpu.VMEM((b_per_w,), jnp.int32),
            pltpu.VMEM((R, G, D), jnp.float32),
            pltpu.SemaphoreType.DMA((R,)),
        ],
        compiler_params=pltpu.CompilerParams(use_tc_tiling_on_sc=False),
    )
    def k(x_hbm, table_hbm, out_hbm, idx_v, rows_v, gsem):
        wid = lax.axis_index("s") * NC + lax.axis_index("c")
        base = wid * b_per_w
        pltpu.sync_copy(x_hbm.at[pl.ds(base, b_per_w)], idx_v)

        def gather(g, b):
            off = pl.multiple_of(g * G, G)
            return pltpu.make_async_copy(
                table_hbm.at[idx_v.at[pl.ds(off, G)]],
                rows_v.at[b],
                gsem.at[b],
            )

        for b in range(R):
            gather(b, b).start()

        def block(o, carry):
            g0 = o * R
            for b in range(R):
                g = g0 + b
                gather(g, b).wait()
                pltpu.sync_copy(rows_v.at[b], out_hbm.at[pl.ds(base + g * G, G)])
                gather(g + R, b).start()
            return carry

        lax.fori_loop(0, n_blocks - 1, block, 0)

        g0 = (n_blocks - 1) * R
        for b in range(R):
            g = g0 + b
            gather(g, b).wait()
            pltpu.sync_copy(rows_v.at[b], out_hbm.at[pl.ds(base + g * G, G)])

    return k


def kernel(x, table):
    lead = x.shape
    xf = x.reshape(-1).astype(jnp.int32)
    out = _make_kernel(xf.shape[0], table.shape[0])(xf, table)
    return out.reshape(*lead, D)


# speedup vs baseline: 1.7940x; 1.6120x over previous
import functools

import jax
import jax.numpy as jnp
from jax import lax
from jax.experimental import pallas as pl
from jax.experimental.pallas import tpu as pltpu
from jax.experimental.pallas import tpu_sc as plsc

NC = 2
NS = 16
R = 8


@functools.lru_cache(maxsize=None)
def _make_kernel(B, S, V, D):
    NW = NC * NS
    rows_per_w = B // NW
    n_blocks = rows_per_w // R
    mesh = plsc.VectorSubcoreMesh(core_axis_name="c", subcore_axis_name="s")

    @functools.partial(
        pl.kernel,
        out_type=jax.ShapeDtypeStruct((B, S, D), jnp.float32),
        mesh=mesh,
        scratch_types=[
            pltpu.VMEM((rows_per_w, S), jnp.int32),
            pltpu.VMEM((R, S, D), jnp.float32),
            pltpu.SemaphoreType.DMA((R,)),
        ],
        compiler_params=pltpu.CompilerParams(use_tc_tiling_on_sc=False),
    )
    def k(x_hbm, table_hbm, out_hbm, idx_v, rows_v, gsem):
        wid = lax.axis_index("s") * NC + lax.axis_index("c")
        row0 = wid * rows_per_w
        pltpu.sync_copy(x_hbm.at[pl.ds(row0, rows_per_w)], idx_v)

        def gather(j, b):
            return pltpu.make_async_copy(
                table_hbm.at[idx_v.at[j]], rows_v.at[b], gsem.at[b]
            )

        for b in range(R):
            gather(b, b).start()

        def block(o, carry):
            j0 = o * R
            for b in range(R):
                j = j0 + b
                gather(j, b).wait()
                pltpu.sync_copy(rows_v.at[b], out_hbm.at[row0 + j])
                gather(j + R, b).start()
            return carry

        lax.fori_loop(0, n_blocks - 1, block, 0)

        j0 = (n_blocks - 1) * R
        for b in range(R):
            j = j0 + b
            gather(j, b).wait()
            pltpu.sync_copy(rows_v.at[b], out_hbm.at[row0 + j])

    return k


def kernel(x, table):
    B, S = x.shape
    V, D = table.shape
    return _make_kernel(B, S, V, D)(x.astype(jnp.int32), table)


# speedup vs baseline: 2.5148x; 1.4018x over previous
import functools

import jax
import jax.numpy as jnp
from jax import lax
from jax.experimental import pallas as pl
from jax.experimental.pallas import tpu as pltpu
from jax.experimental.pallas import tpu_sc as plsc

NC = 2
NS = 16
R = 8


@functools.lru_cache(maxsize=None)
def _make_kernel(B, S, V, D):
    NW = NC * NS
    rows_per_w = B // NW
    n_blocks = rows_per_w // R
    mesh = plsc.VectorSubcoreMesh(core_axis_name="c", subcore_axis_name="s")

    SP = (S + 7) // 8 * 8
    LP = 128

    @functools.partial(
        pl.kernel,
        out_type=jax.ShapeDtypeStruct((B, SP, LP), jnp.float32),
        mesh=mesh,
        scratch_types=[
            pltpu.VMEM((rows_per_w, S), jnp.int32),
            pltpu.VMEM((R, S, D), jnp.float32),
            pltpu.SemaphoreType.DMA((R,)),
        ],
        compiler_params=pltpu.CompilerParams(use_tc_tiling_on_sc=False),
    )
    def k(x_hbm, table_hbm, out_hbm, idx_v, rows_v, gsem):
        wid = lax.axis_index("s") * NC + lax.axis_index("c")
        row0 = wid * rows_per_w
        pltpu.sync_copy(x_hbm.at[pl.ds(row0, rows_per_w)], idx_v)

        def gather(j, b):
            return pltpu.make_async_copy(
                table_hbm.at[idx_v.at[j]], rows_v.at[b], gsem.at[b]
            )

        for b in range(R):
            gather(b, b).start()

        def block(o, carry):
            j0 = o * R
            for b in range(R):
                j = j0 + b
                gather(j, b).wait()
                pltpu.sync_copy(
                    rows_v.at[b],
                    out_hbm.at[row0 + j, pl.ds(0, S), pl.ds(0, D)],
                )
                gather(j + R, b).start()
            return carry

        lax.fori_loop(0, n_blocks - 1, block, 0)

        j0 = (n_blocks - 1) * R
        for b in range(R):
            j = j0 + b
            gather(j, b).wait()
            pltpu.sync_copy(
                rows_v.at[b],
                out_hbm.at[row0 + j, pl.ds(0, S), pl.ds(0, D)],
            )

    return k


def kernel(x, table):
    B, S = x.shape
    V, D = table.shape
    out_p = _make_kernel(B, S, V, D)(x.astype(jnp.int32), table)
    return lax.slice(out_p, (0, 0, 0), (B, S, D))
